# baseline (speedup 1.0000x reference)
import jax
import jax.numpy as jnp
from jax.experimental import pallas as pl
from jax.experimental.pallas import tpu as pltpu

_EPS = 1e-8
_BQ = 256


_LOG2E = 1.4426950408889634


def _body(t_ref, ix_ref, iy_ref, s_ref, qx_ref, qy_ref, o_ref):
    inv_t = 1.0 / (t_ref[0, 0] + _EPS)
    s = s_ref[...]
    m = (s - jnp.min(s)) / (jnp.max(s) - jnp.min(s))
    ix = ix_ref[...]
    iy = iy_ref[...]
    i_norm = ix * ix + iy * iy
    ixb2 = (ix.astype(jnp.bfloat16) * jnp.bfloat16(2.0))
    iyb2 = (iy.astype(jnp.bfloat16) * jnp.bfloat16(2.0))
    n1 = i_norm.astype(jnp.bfloat16)
    rem = i_norm - n1.astype(jnp.float32)
    n2 = rem.astype(jnp.bfloat16)
    n3 = (rem - n2.astype(jnp.float32)).astype(jnp.bfloat16)
    rhs = jnp.concatenate([ixb2, iyb2, -n1, -n2, -n3], axis=0)
    qxb = qx_ref[...].astype(jnp.bfloat16)
    qyb = qy_ref[...].astype(jnp.bfloat16)
    one = jnp.ones_like(qxb)
    lhs = jnp.concatenate([qxb, qyb, one, one, one], axis=1)
    r = jax.lax.dot_general(
        lhs, rhs, (((1,), (0,)), ((), ())),
        preferred_element_type=jnp.float32)
    rmax = jnp.max(r, axis=1, keepdims=True)
    kk = inv_t * _LOG2E
    e = jnp.exp2((r - rmax) * kk)
    z = jnp.sum(e, axis=1, keepdims=True)
    num = jnp.sum(jnp.where(e >= 0.001 * z, e * m, 0.0),
                  axis=1, keepdims=True)
    o_ref[...] = num / z


def kernel(query_vectors, temperature, indices, sofa_metric):
    batch, _ = query_vectors.shape
    db, _ = indices.shape
    orig_dtype = query_vectors.dtype
    q = query_vectors.astype(jnp.float32)
    qx = q[:, 0:1]
    qy = q[:, 1:2]
    ind = indices.astype(jnp.float32)
    ix = ind[:, 0].reshape(1, db)
    iy = ind[:, 1].reshape(1, db)
    s = sofa_metric.astype(jnp.float32).reshape(1, db)
    t = temperature.astype(jnp.float32).reshape(1, 1)

    grid = batch // _BQ
    full = lambda i: (0, 0)
    rows = lambda i: (i, 0)
    out = pl.pallas_call(
        _body,
        grid=(grid,),
        in_specs=[
            pl.BlockSpec((1, 1), full),
            pl.BlockSpec((1, db), full),
            pl.BlockSpec((1, db), full),
            pl.BlockSpec((1, db), full),
            pl.BlockSpec((_BQ, 1), rows),
            pl.BlockSpec((_BQ, 1), rows),
        ],
        out_specs=pl.BlockSpec((_BQ, 1), rows),
        out_shape=jax.ShapeDtypeStruct((batch, 1), jnp.float32),
        compiler_params=pltpu.CompilerParams(
            dimension_semantics=("arbitrary",),
        ),
    )(t, ix, iy, s, qx, qy)
    return out.reshape(batch).astype(orig_dtype)


# speedup vs baseline: 1.0600x; 1.0600x over previous
import jax
import jax.numpy as jnp
from jax.experimental import pallas as pl
from jax.experimental.pallas import tpu as pltpu

_EPS = 1e-8
_BQ = 256


_LOG2E = 1.4426950408889634


def _body(t_ref, ix_ref, iy_ref, s_ref, qx_ref, qy_ref, o_ref):
    inv_t = 1.0 / (t_ref[0, 0] + _EPS)
    s = s_ref[...]
    m = (s - jnp.min(s)) / (jnp.max(s) - jnp.min(s))
    ix = ix_ref[...]
    iy = iy_ref[...]
    i_norm = ix * ix + iy * iy
    ixb2 = (ix.astype(jnp.bfloat16) * jnp.bfloat16(2.0))
    iyb2 = (iy.astype(jnp.bfloat16) * jnp.bfloat16(2.0))
    n1 = i_norm.astype(jnp.bfloat16)
    rem = i_norm - n1.astype(jnp.float32)
    n2 = rem.astype(jnp.bfloat16)
    n3 = (rem - n2.astype(jnp.float32)).astype(jnp.bfloat16)
    rhs = jnp.concatenate([ixb2, iyb2, -n1, -n2, -n3], axis=0)
    qxb = qx_ref[...].astype(jnp.bfloat16)
    qyb = qy_ref[...].astype(jnp.bfloat16)
    one = jnp.ones_like(qxb)
    lhs = jnp.concatenate([qxb, qyb, one, one, one], axis=1)
    r = jax.lax.dot_general(
        lhs, rhs, (((1,), (0,)), ((), ())),
        preferred_element_type=jnp.float32)
    rmax = jnp.max(r, axis=1, keepdims=True)
    kk = inv_t * _LOG2E
    db = r.shape[1]
    chunk = 2048
    es = []
    z = jnp.zeros((r.shape[0], 1), jnp.float32)
    for c0 in range(0, db, chunk):
        ec = jnp.exp2((r[:, c0:c0 + chunk] - rmax) * kk)
        es.append(ec)
        z = z + jnp.sum(ec, axis=1, keepdims=True)
    thr = 0.001 * z
    num = jnp.zeros((r.shape[0], 1), jnp.float32)
    for i, c0 in enumerate(range(0, db, chunk)):
        num = num + jnp.sum(
            jnp.where(es[i] >= thr, es[i] * m[:, c0:c0 + chunk], 0.0),
            axis=1, keepdims=True)
    o_ref[...] = num / z


def kernel(query_vectors, temperature, indices, sofa_metric):
    batch, _ = query_vectors.shape
    db, _ = indices.shape
    orig_dtype = query_vectors.dtype
    q = query_vectors.astype(jnp.float32)
    qx = q[:, 0:1]
    qy = q[:, 1:2]
    ind = indices.astype(jnp.float32)
    ix = ind[:, 0].reshape(1, db)
    iy = ind[:, 1].reshape(1, db)
    s = sofa_metric.astype(jnp.float32).reshape(1, db)
    t = temperature.astype(jnp.float32).reshape(1, 1)

    grid = batch // _BQ
    full = lambda i: (0, 0)
    rows = lambda i: (i, 0)
    out = pl.pallas_call(
        _body,
        grid=(grid,),
        in_specs=[
            pl.BlockSpec((1, 1), full),
            pl.BlockSpec((1, db), full),
            pl.BlockSpec((1, db), full),
            pl.BlockSpec((1, db), full),
            pl.BlockSpec((_BQ, 1), rows),
            pl.BlockSpec((_BQ, 1), rows),
        ],
        out_specs=pl.BlockSpec((_BQ, 1), rows),
        out_shape=jax.ShapeDtypeStruct((batch, 1), jnp.float32),
        compiler_params=pltpu.CompilerParams(
            dimension_semantics=("arbitrary",),
        ),
    )(t, ix, iy, s, qx, qy)
    return out.reshape(batch).astype(orig_dtype)
